# baseline (speedup 1.0000x reference)
import functools

import jax
import jax.numpy as jnp
from jax import lax
from jax.experimental import pallas as pl
from jax.experimental.pallas import tpu as pltpu
from jax.experimental.pallas import tpu_sc as plsc

L = 16
NW = 32
ROW = 128
CHUNK_ROWS = 8


def _make_kernel(n_obs, n_points, n_cams):
    assert n_obs % ROW == 0
    n_rows = n_obs // ROW
    rows_base = n_rows // NW
    rows_extra = n_rows % NW
    chunk_obs = CHUNK_ROWS * ROW
    groups_per_chunk = chunk_obs // L

    mesh = plsc.VectorSubcoreMesh(core_axis_name="c", subcore_axis_name="s")

    @functools.partial(
        pl.kernel,
        mesh=mesh,
        compiler_params=pltpu.CompilerParams(needs_layout_passes=False),
        out_type=jax.ShapeDtypeStruct((n_obs * 2,), jnp.float32),
        scratch_types=[
            pltpu.VMEM((n_cams * 10,), jnp.float32),
            pltpu.VMEM((chunk_obs,), jnp.int32),
            pltpu.VMEM((chunk_obs,), jnp.int32),
            pltpu.VMEM((chunk_obs,), jnp.int32),
            pltpu.VMEM((chunk_obs,), jnp.int32),
            pltpu.VMEM((chunk_obs,), jnp.int32),
            pltpu.VMEM((chunk_obs,), jnp.float32),
            pltpu.VMEM((chunk_obs,), jnp.float32),
            pltpu.VMEM((chunk_obs,), jnp.float32),
            pltpu.VMEM((chunk_obs * 2,), jnp.float32),
            pltpu.SemaphoreType.DMA,
        ],
    )
    def residual_kernel(cidx_hbm, pidx_hbm, pts_hbm, cam_hbm, out_hbm,
                        cam_v, pidx_v, cidx_v, ix_v, iy_v, iz_v,
                        px_v, py_v, pz_v, out_v, sem):
        w = lax.axis_index("s") * 2 + lax.axis_index("c")
        my_rows = rows_base + jnp.where(w < rows_extra, 1, 0)
        row_base = rows_base * w + jnp.minimum(w, rows_extra)
        nchunks = (my_rows + CHUNK_ROWS - 1) // CHUNK_ROWS

        pltpu.sync_copy(cam_hbm, cam_v)

        iota = lax.iota(jnp.int32, L)
        two = jnp.float32(2.0)

        def do_chunk(c, carry):
            rb = row_base + jnp.minimum(c * CHUNK_ROWS, my_rows - CHUNK_ROWS)
            ob = rb * ROW

            pltpu.sync_copy(pidx_hbm.at[pl.ds(ob, chunk_obs)], pidx_v)
            pltpu.sync_copy(cidx_hbm.at[pl.ds(ob, chunk_obs)], cidx_v)

            def prep(g, carry2):
                sl = pl.ds(g * L, L)
                p3 = pidx_v[sl] * 3
                ix_v[sl] = p3
                iy_v[sl] = p3 + 1
                iz_v[sl] = p3 + 2
                return carry2

            lax.fori_loop(0, groups_per_chunk, prep, 0)

            handles = []
            for j in range(CHUNK_ROWS):
                sl = pl.ds(j * ROW, ROW)
                handles.append(
                    pltpu.async_copy(pts_hbm.at[ix_v.at[sl]],
                                     px_v.at[sl], sem))
                handles.append(
                    pltpu.async_copy(pts_hbm.at[iy_v.at[sl]],
                                     py_v.at[sl], sem))
                handles.append(
                    pltpu.async_copy(pts_hbm.at[iz_v.at[sl]],
                                     pz_v.at[sl], sem))
            for h in handles:
                h.wait()

            def do_group(g, carry2):
                sl = pl.ds(g * L, L)
                rows2 = (g * L + iota) * 2
                ci10 = cidx_v[sl] * 10

                px = px_v[sl]
                py = py_v[sl]
                pz = pz_v[sl]

                t0 = plsc.load_gather(cam_v, [ci10])
                t1 = plsc.load_gather(cam_v, [ci10 + 1])
                t2 = plsc.load_gather(cam_v, [ci10 + 2])
                qx = plsc.load_gather(cam_v, [ci10 + 3])
                qy = plsc.load_gather(cam_v, [ci10 + 4])
                qz = plsc.load_gather(cam_v, [ci10 + 5])
                qw = plsc.load_gather(cam_v, [ci10 + 6])
                fo = plsc.load_gather(cam_v, [ci10 + 7])
                k1 = plsc.load_gather(cam_v, [ci10 + 8])
                k2 = plsc.load_gather(cam_v, [ci10 + 9])

                uvx = qy * pz - qz * py
                uvy = qz * px - qx * pz
                uvz = qx * py - qy * px
                uuvx = qy * uvz - qz * uvy
                uuvy = qz * uvx - qx * uvz
                uuvz = qx * uvy - qy * uvx
                cpx = px + two * (qw * uvx + uuvx) + t0
                cpy = py + two * (qw * uvy + uuvy) + t1
                cpz = pz + two * (qw * uvz + uuvz) + t2

                inv = jnp.float32(-1.0) / cpz
                nx = cpx * inv
                ny = cpy * inv
                r2 = nx * nx + ny * ny
                dist = jnp.float32(1.0) + r2 * (k1 + r2 * k2)
                fd = fo * dist

                plsc.store_scatter(out_v, [rows2], fd * nx)
                plsc.store_scatter(out_v, [rows2 + 1], fd * ny)
                return carry2

            lax.fori_loop(0, groups_per_chunk, do_group, 0)
            pltpu.sync_copy(out_v, out_hbm.at[pl.ds(ob * 2, chunk_obs * 2)])
            return carry

        lax.fori_loop(0, nchunks, do_chunk, 0)

    return residual_kernel


def kernel(observes, cidx, pidx, points, camera_params):
    n_obs = observes.shape[0]
    n_points, _ = points.shape
    n_cams, _ = camera_params.shape
    fn = _make_kernel(n_obs, n_points, n_cams)
    proj = fn(cidx.astype(jnp.int32), pidx.astype(jnp.int32),
              points.reshape(-1), camera_params.reshape(-1))
    return proj.reshape(n_obs, 2) - observes


# speedup vs baseline: 8.0235x; 8.0235x over previous
import functools

import jax
import jax.numpy as jnp
from jax import lax
from jax.experimental import pallas as pl
from jax.experimental.pallas import tpu as pltpu
from jax.experimental.pallas import tpu_sc as plsc

L = 16
NW = 32
ROW = 128
CHUNK_ROWS = 8


def _make_kernel(n_obs, n_points, n_cams):
    assert n_obs % ROW == 0
    n_rows = n_obs // ROW
    rows_base = n_rows // NW
    rows_extra = n_rows % NW
    chunk_obs = CHUNK_ROWS * ROW
    groups_per_chunk = chunk_obs // L

    mesh = plsc.VectorSubcoreMesh(core_axis_name="c", subcore_axis_name="s")

    @functools.partial(
        pl.kernel,
        mesh=mesh,
        compiler_params=pltpu.CompilerParams(needs_layout_passes=False),
        out_type=(jax.ShapeDtypeStruct((n_obs,), jnp.float32),
                  jax.ShapeDtypeStruct((n_obs,), jnp.float32)),
        scratch_types=[
            pltpu.VMEM((n_cams * 10,), jnp.float32),
            pltpu.VMEM((chunk_obs,), jnp.int32),
            pltpu.VMEM((chunk_obs,), jnp.int32),
            pltpu.VMEM((chunk_obs,), jnp.float32),
            pltpu.VMEM((chunk_obs,), jnp.float32),
            pltpu.VMEM((chunk_obs,), jnp.float32),
            pltpu.VMEM((chunk_obs,), jnp.float32),
            pltpu.VMEM((chunk_obs,), jnp.float32),
            pltpu.SemaphoreType.DMA,
        ],
    )
    def residual_kernel(cidx_hbm, pidx_hbm, ptx_hbm, pty_hbm, ptz_hbm,
                        cam_hbm, outx_hbm, outy_hbm,
                        cam_v, pidx_v, cidx_v, px_v, py_v, pz_v,
                        ox_v, oy_v, sem):
        w = lax.axis_index("s") * 2 + lax.axis_index("c")
        my_rows = rows_base + jnp.where(w < rows_extra, 1, 0)
        row_base = rows_base * w + jnp.minimum(w, rows_extra)
        nchunks = (my_rows + CHUNK_ROWS - 1) // CHUNK_ROWS

        pltpu.sync_copy(cam_hbm, cam_v)

        two = jnp.float32(2.0)

        def do_chunk(c, carry):
            rb = row_base + jnp.minimum(c * CHUNK_ROWS, my_rows - CHUNK_ROWS)
            ob = rb * ROW

            pltpu.sync_copy(pidx_hbm.at[pl.ds(ob, chunk_obs)], pidx_v)
            pltpu.sync_copy(cidx_hbm.at[pl.ds(ob, chunk_obs)], cidx_v)

            handles = []
            for j in range(CHUNK_ROWS):
                sl = pl.ds(j * ROW, ROW)
                idx = pidx_v.at[sl]
                handles.append(
                    pltpu.async_copy(ptx_hbm.at[idx], px_v.at[sl], sem))
                handles.append(
                    pltpu.async_copy(pty_hbm.at[idx], py_v.at[sl], sem))
                handles.append(
                    pltpu.async_copy(ptz_hbm.at[idx], pz_v.at[sl], sem))
            for h in handles:
                h.wait()

            def do_group(g, carry2):
                sl = pl.ds(g * L, L)
                ci10 = cidx_v[sl] * 10

                px = px_v[sl]
                py = py_v[sl]
                pz = pz_v[sl]

                t0 = plsc.load_gather(cam_v, [ci10])
                t1 = plsc.load_gather(cam_v, [ci10 + 1])
                t2 = plsc.load_gather(cam_v, [ci10 + 2])
                qx = plsc.load_gather(cam_v, [ci10 + 3])
                qy = plsc.load_gather(cam_v, [ci10 + 4])
                qz = plsc.load_gather(cam_v, [ci10 + 5])
                qw = plsc.load_gather(cam_v, [ci10 + 6])
                fo = plsc.load_gather(cam_v, [ci10 + 7])
                k1 = plsc.load_gather(cam_v, [ci10 + 8])
                k2 = plsc.load_gather(cam_v, [ci10 + 9])

                uvx = qy * pz - qz * py
                uvy = qz * px - qx * pz
                uvz = qx * py - qy * px
                uuvx = qy * uvz - qz * uvy
                uuvy = qz * uvx - qx * uvz
                uuvz = qx * uvy - qy * uvx
                cpx = px + two * (qw * uvx + uuvx) + t0
                cpy = py + two * (qw * uvy + uuvy) + t1
                cpz = pz + two * (qw * uvz + uuvz) + t2

                inv = jnp.float32(-1.0) / cpz
                nx = cpx * inv
                ny = cpy * inv
                r2 = nx * nx + ny * ny
                dist = jnp.float32(1.0) + r2 * (k1 + r2 * k2)
                fd = fo * dist

                ox_v[sl] = fd * nx
                oy_v[sl] = fd * ny
                return carry2

            lax.fori_loop(0, groups_per_chunk, do_group, 0)
            pltpu.sync_copy(ox_v, outx_hbm.at[pl.ds(ob, chunk_obs)])
            pltpu.sync_copy(oy_v, outy_hbm.at[pl.ds(ob, chunk_obs)])
            return carry

        lax.fori_loop(0, nchunks, do_chunk, 0)

    return residual_kernel


def kernel(observes, cidx, pidx, points, camera_params):
    n_obs = observes.shape[0]
    n_points, _ = points.shape
    n_cams, _ = camera_params.shape
    fn = _make_kernel(n_obs, n_points, n_cams)
    proj_x, proj_y = fn(cidx.astype(jnp.int32), pidx.astype(jnp.int32),
                        points[:, 0], points[:, 1], points[:, 2],
                        camera_params.reshape(-1))
    return jnp.stack([proj_x, proj_y], axis=-1) - observes


# speedup vs baseline: 12.5601x; 1.5654x over previous
import functools

import jax
import jax.numpy as jnp
from jax import lax
from jax.experimental import pallas as pl
from jax.experimental.pallas import tpu as pltpu
from jax.experimental.pallas import tpu_sc as plsc

L = 16
NW = 32
ROW = 128
CHUNK_ROWS = 8
NPHASE = 64


def _make_kernel(n_obs, n_points, n_cams):
    assert n_obs % ROW == 0
    n_rows = n_obs // ROW
    rows_base = n_rows // NW
    rows_extra = n_rows % NW
    chunk_obs = CHUNK_ROWS * ROW
    groups_per_chunk = chunk_obs // L
    assert (rows_base + 1 + CHUNK_ROWS - 1) // CHUNK_ROWS <= NPHASE

    mesh = plsc.VectorSubcoreMesh(core_axis_name="c", subcore_axis_name="s")

    buf_t = [
        pltpu.VMEM((chunk_obs,), jnp.int32),
        pltpu.VMEM((chunk_obs,), jnp.int32),
        pltpu.VMEM((chunk_obs,), jnp.float32),
        pltpu.VMEM((chunk_obs,), jnp.float32),
        pltpu.VMEM((chunk_obs,), jnp.float32),
        pltpu.VMEM((chunk_obs,), jnp.float32),
        pltpu.VMEM((chunk_obs,), jnp.float32),
        pltpu.SemaphoreType.DMA,
        pltpu.SemaphoreType.DMA,
        pltpu.SemaphoreType.DMA,
    ]

    @functools.partial(
        pl.kernel,
        mesh=mesh,
        compiler_params=pltpu.CompilerParams(needs_layout_passes=False),
        out_type=(jax.ShapeDtypeStruct((n_obs,), jnp.float32),
                  jax.ShapeDtypeStruct((n_obs,), jnp.float32)),
        scratch_types=[pltpu.VMEM((n_cams * 10,), jnp.float32)] + buf_t * 2,
    )
    def residual_kernel(cidx_hbm, pidx_hbm, ptx_hbm, pty_hbm, ptz_hbm,
                        cam_hbm, outx_hbm, outy_hbm, cam_v, *bufs):
        A, B = bufs[:10], bufs[10:]
        w = lax.axis_index("s") * 2 + lax.axis_index("c")
        my_rows = rows_base + jnp.where(w < rows_extra, 1, 0)
        row_base = rows_base * w + jnp.minimum(w, rows_extra)

        pltpu.sync_copy(cam_hbm, cam_v)

        two = jnp.float32(2.0)

        def base_ob(c):
            rb = row_base + jnp.minimum(c * CHUNK_ROWS, my_rows - CHUNK_ROWS)
            return rb * ROW

        def lin_issue(c, b):
            ob = base_ob(c)
            pltpu.async_copy(pidx_hbm.at[pl.ds(ob, chunk_obs)], b[0], b[7])
            pltpu.async_copy(cidx_hbm.at[pl.ds(ob, chunk_obs)], b[1], b[7])

        def lin_wait(b):
            pltpu.make_async_copy(
                pidx_hbm.at[pl.ds(0, chunk_obs)], b[0], b[7]).wait()
            pltpu.make_async_copy(
                cidx_hbm.at[pl.ds(0, chunk_obs)], b[1], b[7]).wait()

        def gather_fire(b):
            for j in range(CHUNK_ROWS):
                sl = pl.ds(j * ROW, ROW)
                idx = b[0].at[sl]
                pltpu.async_copy(ptx_hbm.at[idx], b[2].at[sl], b[8])
                pltpu.async_copy(pty_hbm.at[idx], b[3].at[sl], b[8])
                pltpu.async_copy(ptz_hbm.at[idx], b[4].at[sl], b[8])

        def gather_wait(b):
            for j in range(CHUNK_ROWS):
                sl = pl.ds(j * ROW, ROW)
                for t in (2, 3, 4):
                    pltpu.make_async_copy(
                        ptx_hbm.at[pl.ds(0, ROW)], b[t].at[sl], b[8]).wait()

        def out_issue(c, b):
            ob = base_ob(c)
            pltpu.async_copy(b[5], outx_hbm.at[pl.ds(ob, chunk_obs)], b[9])
            pltpu.async_copy(b[6], outy_hbm.at[pl.ds(ob, chunk_obs)], b[9])

        def out_wait(b):
            pltpu.make_async_copy(
                b[5], outx_hbm.at[pl.ds(0, chunk_obs)], b[9]).wait()
            pltpu.make_async_copy(
                b[6], outy_hbm.at[pl.ds(0, chunk_obs)], b[9]).wait()

        def compute_chunk(b):
            def do_group(g, carry):
                sl = pl.ds(g * L, L)
                ci10 = b[1][sl] * 10

                px = b[2][sl]
                py = b[3][sl]
                pz = b[4][sl]

                t0 = plsc.load_gather(cam_v, [ci10])
                t1 = plsc.load_gather(cam_v, [ci10 + 1])
                t2 = plsc.load_gather(cam_v, [ci10 + 2])
                qx = plsc.load_gather(cam_v, [ci10 + 3])
                qy = plsc.load_gather(cam_v, [ci10 + 4])
                qz = plsc.load_gather(cam_v, [ci10 + 5])
                qw = plsc.load_gather(cam_v, [ci10 + 6])
                fo = plsc.load_gather(cam_v, [ci10 + 7])
                k1 = plsc.load_gather(cam_v, [ci10 + 8])
                k2 = plsc.load_gather(cam_v, [ci10 + 9])

                uvx = qy * pz - qz * py
                uvy = qz * px - qx * pz
                uvz = qx * py - qy * px
                uuvx = qy * uvz - qz * uvy
                uuvy = qz * uvx - qx * uvz
                uuvz = qx * uvy - qy * uvx
                cpx = px + two * (qw * uvx + uuvx) + t0
                cpy = py + two * (qw * uvy + uuvy) + t1
                cpz = pz + two * (qw * uvz + uuvz) + t2

                inv = jnp.float32(-1.0) / cpz
                nx = cpx * inv
                ny = cpy * inv
                r2 = nx * nx + ny * ny
                dist = jnp.float32(1.0) + r2 * (k1 + r2 * k2)
                fd = fo * dist

                b[5][sl] = fd * nx
                b[6][sl] = fd * ny
                return carry

            lax.fori_loop(0, groups_per_chunk, do_group, 0)

        def phase(c, cur, nxt, wait_out, comp):
            lin_wait(cur)
            gather_fire(cur)
            if wait_out:
                out_wait(nxt)
            if comp:
                gather_wait(nxt)
                lin_issue(c + 1, nxt)
                compute_chunk(nxt)
                out_issue(c - 1, nxt)
            else:
                lin_issue(c + 1, nxt)

        lin_issue(0, A)
        phase(jnp.int32(0), A, B, False, False)
        phase(jnp.int32(1), B, A, False, True)
        phase(jnp.int32(2), A, B, False, True)
        phase(jnp.int32(3), B, A, True, True)

        def pair(i, carry):
            c = 2 * i
            phase(c, A, B, True, True)
            phase(c + 1, B, A, True, True)
            return carry

        lax.fori_loop(2, NPHASE // 2, pair, 0)

        lin_wait(A)
        out_wait(B)
        gather_wait(B)
        compute_chunk(B)
        out_issue(jnp.int32(NPHASE - 1), B)
        out_wait(A)
        out_wait(B)

    return residual_kernel


def kernel(observes, cidx, pidx, points, camera_params):
    n_obs = observes.shape[0]
    n_points, _ = points.shape
    n_cams, _ = camera_params.shape
    fn = _make_kernel(n_obs, n_points, n_cams)
    proj_x, proj_y = fn(cidx.astype(jnp.int32), pidx.astype(jnp.int32),
                        points[:, 0], points[:, 1], points[:, 2],
                        camera_params.reshape(-1))
    return jnp.stack([proj_x, proj_y], axis=-1) - observes
